# baseline (speedup 1.0000x reference)
import functools
import jax
import jax.numpy as jnp
from jax import lax
from jax.experimental import pallas as pl
from jax.experimental.pallas import tpu as pltpu
from jax.experimental.pallas import tpu_sc as plsc

N = 100000
E = 1600000
ROW_BLK = 4000
CHUNK = 400
NS = 16
ROWS_PER_TILE = 6256
ROWS_LAST = N - 15 * ROWS_PER_TILE
WB_CHUNK = 400

_MESH = plsc.VectorSubcoreMesh(core_axis_name="c", subcore_axis_name="s")
_SC_PARAMS = pltpu.CompilerParams(use_tc_tiling_on_sc=False)


def _encode_body(veh_x, pas_x, W_veh, b_veh, W_pas, b_pas, veh_o, pas_o):
    veh_o[...] = jnp.tanh(
        jnp.dot(veh_x[...], W_veh[...], preferred_element_type=jnp.float32) + b_veh[...])
    pas_o[...] = jnp.tanh(
        jnp.dot(pas_x[...], W_pas[...], preferred_element_type=jnp.float32) + b_pas[...])


def _encode(veh_x, pas_x, W_veh, b_veh, W_pas, b_pas):
    n = veh_x.shape[0]
    rows = lambda w: pl.BlockSpec((ROW_BLK, w), lambda i: (i, 0))
    full = lambda a: pl.BlockSpec(a.shape, lambda i: (0,) * a.ndim)
    return pl.pallas_call(
        _encode_body,
        grid=(n // ROW_BLK,),
        in_specs=[rows(8), rows(10),
                  full(W_veh), full(b_veh), full(W_pas), full(b_pas)],
        out_specs=[rows(16), rows(16)],
        out_shape=[jax.ShapeDtypeStruct((n, 16), jnp.float32)] * 2,
    )(veh_x, pas_x, W_veh, b_veh, W_pas, b_pas)


def _merge_body(acc, out):
    out[...] = acc[0] + acc[1]


def _merge(acc):
    n = acc.shape[1]
    return pl.pallas_call(
        _merge_body,
        grid=(n // ROW_BLK,),
        in_specs=[pl.BlockSpec((2, ROW_BLK, 16), lambda i: (0, i, 0))],
        out_specs=pl.BlockSpec((ROW_BLK, 16), lambda i: (i, 0)),
        out_shape=jax.ShapeDtypeStruct((n, 16), jnp.float32),
    )(acc)


def _head_body(req_x, lo, hi, W_req, b_req, W1, b1, W2, b2, W3, b3, out):
    req = jnp.tanh(
        jnp.dot(req_x[...], W_req[...], preferred_element_type=jnp.float32) + b_req[...])
    act = jnp.concatenate([req, lo[...], hi[...]], axis=-1)
    h = jnp.tanh(jnp.dot(act, W1[...], preferred_element_type=jnp.float32) + b1[...])
    h = jnp.tanh(jnp.dot(h, W2[...], preferred_element_type=jnp.float32) + b2[...])
    out[...] = jnp.dot(h, W3[...], preferred_element_type=jnp.float32) + b3[...]


def _head(req_x, agg_lo, agg_hi, W_req, b_req, W1, b1, W2, b2, W3, b3):
    n = req_x.shape[0]
    rows = lambda w: pl.BlockSpec((ROW_BLK, w), lambda i: (i, 0))
    full = lambda a: pl.BlockSpec(a.shape, lambda i: (0,) * a.ndim)
    return pl.pallas_call(
        _head_body,
        grid=(n // ROW_BLK,),
        in_specs=[rows(10), rows(16), rows(16),
                  full(W_req), full(b_req),
                  full(W1), full(b1), full(W2), full(b2), full(W3), full(b3)],
        out_specs=rows(1),
        out_shape=jax.ShapeDtypeStruct((n, 1), jnp.float32),
    )(req_x, agg_lo, agg_hi, W_req, b_req, W1, b1, W2, b2, W3, b3)


def _fill(ref, val):
    flat = ref.shape[0]
    v = jnp.full((16,), val, jnp.float32)

    def body(i, _):
        ref[pl.ds(i * 16, 16)] = v
        return 0

    lax.fori_loop(0, flat // 16, body, 0)
    if flat % 16:
        ref[pl.ds(flat - 16, 16)] = v


def _scale_rows(rows, cbuf, ibuf, nrows):
    def inv(i, _):
        ibuf[pl.ds(i * 16, 16)] = 1.0 / jnp.maximum(cbuf[pl.ds(i * 16, 16)], 1.0)
        return 0

    lax.fori_loop(0, nrows // 16, inv, 0)

    def mul16(k, _):
        iv = ibuf[pl.ds(k * 16, 16)]
        for j in range(16):
            rows[k * 16 + j, :] = rows[k * 16 + j, :] * iv[j]
        return 0

    lax.fori_loop(0, nrows // 16, mul16, 0)


def _scale_writeback(acc_sh, cnt_sh, out_acc, c, s, rows, cbuf, ibuf):
    base = pl.multiple_of(s * ROWS_PER_TILE, 8)

    def sub(k, _):
        r = pl.multiple_of(base + k * WB_CHUNK, 8)
        pltpu.sync_copy(acc_sh.at[pl.ds(r, WB_CHUNK)], rows.at[pl.ds(0, WB_CHUNK)])
        pltpu.sync_copy(cnt_sh.at[pl.ds(r, WB_CHUNK)], cbuf.at[pl.ds(0, WB_CHUNK)])
        _scale_rows(rows, cbuf, ibuf, WB_CHUNK)
        pltpu.sync_copy(rows.at[pl.ds(0, WB_CHUNK)], out_acc.at[c, pl.ds(r, WB_CHUNK)])
        return 0

    lax.fori_loop(0, 15, sub, 0)

    def tail(sz):
        r = pl.multiple_of(base + 15 * WB_CHUNK, 8)
        pltpu.sync_copy(acc_sh.at[pl.ds(r, sz)], rows.at[pl.ds(0, sz)])
        pltpu.sync_copy(cnt_sh.at[pl.ds(r, sz)], cbuf.at[pl.ds(0, sz)])
        _scale_rows(rows, cbuf, ibuf, sz)
        pltpu.sync_copy(rows.at[pl.ds(0, sz)], out_acc.at[c, pl.ds(r, sz)])

    @pl.when(s < NS - 1)
    def _():
        tail(ROWS_PER_TILE - 15 * WB_CHUNK)

    @pl.when(s == NS - 1)
    def _():
        tail(ROWS_LAST - 15 * WB_CHUNK)


def _gather_pipe(tbl_for, gidx, sidx, acc_sh, cnt_sh, aux,
                 bufs, nsteps, off_fn):
    (ig0, is0, r0, sem0), (ig1, is1, r1, sem1) = bufs

    def step(j, _):
        off = off_fn(j)
        pltpu.sync_copy(gidx.at[pl.ds(off, CHUNK)], ig0)
        pltpu.sync_copy(sidx.at[pl.ds(off, CHUNK)], is0)
        pltpu.async_copy(tbl_for(ig0), r0, sem0).wait()
        pltpu.sync_copy(r0, acc_sh.at[is0], add=True)
        pltpu.sync_copy(aux, cnt_sh.at[is0], add=True)
        return 0

    lax.fori_loop(0, nsteps, step, 0)


def _count_pipe(sidx, cnt_sh, aux, bufs, nsteps, off_fn):
    (is0, sem0), (is1, sem1) = bufs

    def step(j, _):
        off = off_fn(j)
        pltpu.sync_copy(sidx.at[pl.ds(off, CHUNK)], is0)
        pltpu.sync_copy(aux, cnt_sh.at[is0], add=True)
        return 0

    lax.fori_loop(0, nsteps, step, 0)


def _scatter1_body(tbl, gidx, sidx, z16, z1, out_acc,
                   ig0, is0, r0, ig1, is1, r1, aux, cbuf,
                   acc_sh, cnt_sh, sem0, sem1):
    c = lax.axis_index("c")
    s = lax.axis_index("s")
    _fill(aux, 1.0)

    @pl.when(s == 0)
    def _():
        pltpu.sync_copy(z16, acc_sh)
        pltpu.sync_copy(z1, cnt_sh)
    plsc.subcore_barrier()

    per_tile = E // NS
    feat_half = per_tile // 2
    cnt_base = s * per_tile
    nsteps = feat_half // CHUNK

    feat_base = cnt_base + c * feat_half
    _gather_pipe(lambda ig: tbl.at[ig], gidx, sidx, acc_sh, cnt_sh, aux,
                 [(ig0, is0, r0, sem0), (ig1, is1, r1, sem1)], nsteps,
                 lambda j: pl.multiple_of(feat_base + j * CHUNK, 8))

    cnt_base2 = cnt_base + (1 - c) * feat_half
    _count_pipe(sidx, cnt_sh, aux, [(is0, sem0), (is1, sem1)], nsteps,
                lambda j: pl.multiple_of(cnt_base2 + j * CHUNK, 8))

    plsc.subcore_barrier()
    _scale_writeback(acc_sh, cnt_sh, out_acc, c, s, r0, cbuf, aux)


def _scatter_phase1(pas16, recv, send, z16, z1):
    return pl.kernel(
        _scatter1_body,
        out_type=jax.ShapeDtypeStruct((2, N, 16), jnp.float32),
        mesh=_MESH,
        compiler_params=_SC_PARAMS,
        scratch_types=[
            pltpu.VMEM((CHUNK,), jnp.int32),
            pltpu.VMEM((CHUNK,), jnp.int32),
            pltpu.VMEM((CHUNK, 16), jnp.float32),
            pltpu.VMEM((CHUNK,), jnp.int32),
            pltpu.VMEM((CHUNK,), jnp.int32),
            pltpu.VMEM((CHUNK, 16), jnp.float32),
            pltpu.VMEM((CHUNK,), jnp.float32),
            pltpu.VMEM((CHUNK,), jnp.float32),
            pltpu.VMEM_SHARED((N, 16), jnp.float32),
            pltpu.VMEM_SHARED((N,), jnp.float32),
            pltpu.SemaphoreType.DMA,
            pltpu.SemaphoreType.DMA,
        ],
    )(pas16, recv, send, z16, z1)


def _scatter2_body(tbl_lo, tbl_hi, gidx, sidx, z16, z1, out_acc,
                   ig0, is0, r0, ig1, is1, r1, aux, cbuf,
                   acc_sh, cnt_sh, sem0, sem1):
    c = lax.axis_index("c")
    s = lax.axis_index("s")
    _fill(aux, 1.0)

    @pl.when(s == 0)
    def _():
        pltpu.sync_copy(z16, acc_sh)
        pltpu.sync_copy(z1, cnt_sh)
    plsc.subcore_barrier()

    per_tile = E // NS
    base = s * per_tile
    nsteps = per_tile // CHUNK
    bufs = [(ig0, is0, r0, sem0), (ig1, is1, r1, sem1)]
    off_fn = lambda j: pl.multiple_of(base + j * CHUNK, 8)

    @pl.when(c == 0)
    def _():
        _gather_pipe(lambda ig: tbl_lo.at[ig], gidx, sidx, acc_sh, cnt_sh,
                     aux, bufs, nsteps, off_fn)

    @pl.when(c == 1)
    def _():
        _gather_pipe(lambda ig: tbl_hi.at[ig], gidx, sidx, acc_sh, cnt_sh,
                     aux, bufs, nsteps, off_fn)

    plsc.subcore_barrier()
    _scale_writeback(acc_sh, cnt_sh, out_acc, c, s, r0, cbuf, aux)


def _scatter_phase2(veh16, pas_mean, dest, src, z16, z1):
    return pl.kernel(
        _scatter2_body,
        out_type=jax.ShapeDtypeStruct((2, N, 16), jnp.float32),
        mesh=_MESH,
        compiler_params=_SC_PARAMS,
        scratch_types=[
            pltpu.VMEM((CHUNK,), jnp.int32),
            pltpu.VMEM((CHUNK,), jnp.int32),
            pltpu.VMEM((CHUNK, 16), jnp.float32),
            pltpu.VMEM((CHUNK,), jnp.int32),
            pltpu.VMEM((CHUNK,), jnp.int32),
            pltpu.VMEM((CHUNK, 16), jnp.float32),
            pltpu.VMEM((CHUNK,), jnp.float32),
            pltpu.VMEM((CHUNK,), jnp.float32),
            pltpu.VMEM_SHARED((N, 16), jnp.float32),
            pltpu.VMEM_SHARED((N,), jnp.float32),
            pltpu.SemaphoreType.DMA,
            pltpu.SemaphoreType.DMA,
        ],
    )(veh16, pas_mean, dest, src, z16, z1)


def kernel(requests_x, vehicles_x, passengers_x,
           veh2pas_receiver_edge_index, veh2pas_sender_edge_index,
           req2veh_sender_edge_index, req2veh_receiver_edge_index,
           W_req, b_req, W_veh, b_veh, W_pas, b_pas,
           W1, b1, W2, b2, W3, b3):
    veh16, pas16 = _encode(vehicles_x, passengers_x,
                           W_veh, b_veh.reshape(1, 16),
                           W_pas, b_pas.reshape(1, 16))

    z16 = jnp.zeros((N, 16), jnp.float32)
    z1 = jnp.zeros((N,), jnp.float32)

    meanpart = _scatter_phase1(
        pas16, veh2pas_receiver_edge_index, veh2pas_sender_edge_index, z16, z1)
    pas_mean = _merge(meanpart)

    mean2 = _scatter_phase2(
        veh16, pas_mean, req2veh_receiver_edge_index,
        req2veh_sender_edge_index, z16, z1)

    return _head(requests_x, mean2[0], mean2[1],
                 W_req, b_req.reshape(1, 16),
                 W1, b1.reshape(1, 64), W2, b2.reshape(1, 64),
                 W3, b3.reshape(1, 1))


# speedup vs baseline: 1.5840x; 1.5840x over previous
import functools
import jax
import jax.numpy as jnp
from jax import lax
from jax.experimental import pallas as pl
from jax.experimental.pallas import tpu as pltpu
from jax.experimental.pallas import tpu_sc as plsc

N = 100000
E = 1600000
ROW_BLK = 4000
CHUNK = 1000
NS = 16
ROWS_PER_TILE = 6256
ROWS_LAST = N - 15 * ROWS_PER_TILE
WB_CHUNK = 800

_MESH = plsc.VectorSubcoreMesh(core_axis_name="c", subcore_axis_name="s")
_SC_PARAMS = pltpu.CompilerParams(use_tc_tiling_on_sc=False)


def _encode_body(veh_x, pas_x, W_veh, b_veh, W_pas, b_pas, veh_o, pas_o):
    veh_o[...] = jnp.tanh(
        jnp.dot(veh_x[...], W_veh[...], preferred_element_type=jnp.float32) + b_veh[...])
    pas_o[...] = jnp.tanh(
        jnp.dot(pas_x[...], W_pas[...], preferred_element_type=jnp.float32) + b_pas[...])


def _encode(veh_x, pas_x, W_veh, b_veh, W_pas, b_pas):
    n = veh_x.shape[0]
    rows = lambda w: pl.BlockSpec((ROW_BLK, w), lambda i: (i, 0))
    full = lambda a: pl.BlockSpec(a.shape, lambda i: (0,) * a.ndim)
    return pl.pallas_call(
        _encode_body,
        grid=(n // ROW_BLK,),
        in_specs=[rows(8), rows(10),
                  full(W_veh), full(b_veh), full(W_pas), full(b_pas)],
        out_specs=[rows(16), rows(16)],
        out_shape=[jax.ShapeDtypeStruct((n, 16), jnp.float32)] * 2,
    )(veh_x, pas_x, W_veh, b_veh, W_pas, b_pas)


def _merge_body(a, b, out):
    out[...] = a[0] + b[0]


def _merge(acc):
    flat = acc.reshape(2, 12500, 128)
    out = pl.pallas_call(
        _merge_body,
        grid=(1,),
        in_specs=[pl.BlockSpec((1, 12500, 128), lambda i: (0, 0, 0)),
                  pl.BlockSpec((1, 12500, 128), lambda i: (1, 0, 0))],
        out_specs=pl.BlockSpec((12500, 128), lambda i: (0, 0)),
        out_shape=jax.ShapeDtypeStruct((12500, 128), jnp.float32),
    )(flat, flat)
    return out.reshape(N, 16)


def _head_body(req_x, m2, W_req, b_req, W1, b1, W2, b2, W3, b3, out):
    req = jnp.tanh(
        jnp.dot(req_x[...], W_req[...], preferred_element_type=jnp.float32) + b_req[...])
    act = jnp.concatenate([req, m2[0], m2[1]], axis=-1)
    h = jnp.tanh(jnp.dot(act, W1[...], preferred_element_type=jnp.float32) + b1[...])
    h = jnp.tanh(jnp.dot(h, W2[...], preferred_element_type=jnp.float32) + b2[...])
    out[...] = jnp.dot(h, W3[...], preferred_element_type=jnp.float32) + b3[...]


def _head(req_x, mean2, W_req, b_req, W1, b1, W2, b2, W3, b3):
    n = req_x.shape[0]
    rows = lambda w: pl.BlockSpec((ROW_BLK, w), lambda i: (i, 0))
    full = lambda a: pl.BlockSpec(a.shape, lambda i: (0,) * a.ndim)
    return pl.pallas_call(
        _head_body,
        grid=(n // ROW_BLK,),
        in_specs=[rows(10), pl.BlockSpec((2, ROW_BLK, 16), lambda i: (0, i, 0)),
                  full(W_req), full(b_req),
                  full(W1), full(b1), full(W2), full(b2), full(W3), full(b3)],
        out_specs=rows(1),
        out_shape=jax.ShapeDtypeStruct((n, 1), jnp.float32),
    )(req_x, mean2, W_req, b_req, W1, b1, W2, b2, W3, b3)


def _fill(ref, val):
    flat = ref.shape[0]
    v = jnp.full((16,), val, jnp.float32)

    def body(i, _):
        ref[pl.ds(i * 16, 16)] = v
        return 0

    lax.fori_loop(0, flat // 16, body, 0)
    if flat % 16:
        ref[pl.ds(flat - 16, 16)] = v


def _scale_rows(rows, cbuf, ibuf, nrows):
    def inv(i, _):
        ibuf[pl.ds(i * 16, 16)] = 1.0 / jnp.maximum(cbuf[pl.ds(i * 16, 16)], 1.0)
        return 0

    lax.fori_loop(0, nrows // 16, inv, 0)

    def mul16(k, _):
        iv = ibuf[pl.ds(k * 16, 16)]
        for j in range(16):
            rows[k * 16 + j, :] = rows[k * 16 + j, :] * iv[j]
        return 0

    lax.fori_loop(0, nrows // 16, mul16, 0)


def _scale_writeback(acc_sh, cnt_sh, out_acc, c, s, rows, cbuf, ibuf):
    base = pl.multiple_of(s * ROWS_PER_TILE, 8)

    def sub(k, _):
        r = pl.multiple_of(base + k * WB_CHUNK, 8)
        pltpu.sync_copy(acc_sh.at[pl.ds(r, WB_CHUNK)], rows.at[pl.ds(0, WB_CHUNK)])
        pltpu.sync_copy(cnt_sh.at[pl.ds(r, WB_CHUNK)], cbuf.at[pl.ds(0, WB_CHUNK)])
        _scale_rows(rows, cbuf, ibuf, WB_CHUNK)
        pltpu.sync_copy(rows.at[pl.ds(0, WB_CHUNK)], out_acc.at[c, pl.ds(r, WB_CHUNK)])
        return 0

    lax.fori_loop(0, 7, sub, 0)

    def tail(sz):
        r = pl.multiple_of(base + 7 * WB_CHUNK, 8)
        pltpu.sync_copy(acc_sh.at[pl.ds(r, sz)], rows.at[pl.ds(0, sz)])
        pltpu.sync_copy(cnt_sh.at[pl.ds(r, sz)], cbuf.at[pl.ds(0, sz)])
        _scale_rows(rows, cbuf, ibuf, sz)
        pltpu.sync_copy(rows.at[pl.ds(0, sz)], out_acc.at[c, pl.ds(r, sz)])

    @pl.when(s < NS - 1)
    def _():
        tail(ROWS_PER_TILE - 7 * WB_CHUNK)

    @pl.when(s == NS - 1)
    def _():
        tail(ROWS_LAST - 7 * WB_CHUNK)


def _gather_pipe(tbl_for, gidx, sidx, acc_sh, cnt_sh, aux,
                 bufs, nsteps, off_fn):
    ((ig0, is0, r0, sem0),) = bufs

    def step(j, _):
        off = off_fn(j)
        pltpu.sync_copy(gidx.at[pl.ds(off, CHUNK)], ig0)
        pltpu.sync_copy(sidx.at[pl.ds(off, CHUNK)], is0)
        pltpu.async_copy(tbl_for(ig0), r0, sem0).wait()
        pltpu.sync_copy(r0, acc_sh.at[is0], add=True)
        pltpu.sync_copy(aux, cnt_sh.at[is0], add=True)
        return 0

    lax.fori_loop(0, nsteps, step, 0)


def _count_pipe(sidx, cnt_sh, aux, bufs, nsteps, off_fn):
    ((is0, sem0),) = bufs

    def step(j, _):
        off = off_fn(j)
        pltpu.sync_copy(sidx.at[pl.ds(off, CHUNK)], is0)
        pltpu.sync_copy(aux, cnt_sh.at[is0], add=True)
        return 0

    lax.fori_loop(0, nsteps, step, 0)


def _scatter1_body(tbl, gidx, sidx, z16, z1, out_acc,
                   ig0, is0, r0, aux, cbuf, acc_sh, cnt_sh, sem0):
    c = lax.axis_index("c")
    s = lax.axis_index("s")
    _fill(aux, 1.0)

    @pl.when(s == 0)
    def _():
        pltpu.sync_copy(z16, acc_sh)
        pltpu.sync_copy(z1, cnt_sh)
    plsc.subcore_barrier()

    per_tile = E // NS
    feat_half = per_tile // 2
    cnt_base = s * per_tile
    nsteps = feat_half // CHUNK

    feat_base = cnt_base + c * feat_half
    _gather_pipe(lambda ig: tbl.at[ig], gidx, sidx, acc_sh, cnt_sh, aux,
                 [(ig0, is0, r0, sem0)], nsteps,
                 lambda j: pl.multiple_of(feat_base + j * CHUNK, 8))

    cnt_base2 = cnt_base + (1 - c) * feat_half
    _count_pipe(sidx, cnt_sh, aux, [(is0, sem0)], nsteps,
                lambda j: pl.multiple_of(cnt_base2 + j * CHUNK, 8))

    plsc.subcore_barrier()
    _scale_writeback(acc_sh, cnt_sh, out_acc, c, s, r0, cbuf, aux)


def _scatter_phase1(pas16, recv, send, z16, z1):
    return pl.kernel(
        _scatter1_body,
        out_type=jax.ShapeDtypeStruct((2, N, 16), jnp.float32),
        mesh=_MESH,
        compiler_params=_SC_PARAMS,
        scratch_types=[
            pltpu.VMEM((CHUNK,), jnp.int32),
            pltpu.VMEM((CHUNK,), jnp.int32),
            pltpu.VMEM((CHUNK, 16), jnp.float32),
            pltpu.VMEM((CHUNK,), jnp.float32),
            pltpu.VMEM((CHUNK,), jnp.float32),
            pltpu.VMEM_SHARED((N, 16), jnp.float32),
            pltpu.VMEM_SHARED((N,), jnp.float32),
            pltpu.SemaphoreType.DMA,
        ],
    )(pas16, recv, send, z16, z1)


def _scatter2_body(tbl_lo, tbl_hi, gidx, sidx, z16, z1, out_acc,
                   ig0, is0, r0, aux, cbuf, acc_sh, cnt_sh, sem0):
    c = lax.axis_index("c")
    s = lax.axis_index("s")
    _fill(aux, 1.0)

    @pl.when(s == 0)
    def _():
        pltpu.sync_copy(z16, acc_sh)
        pltpu.sync_copy(z1, cnt_sh)
    plsc.subcore_barrier()

    per_tile = E // NS
    base = s * per_tile
    nsteps = per_tile // CHUNK
    bufs = [(ig0, is0, r0, sem0)]
    off_fn = lambda j: pl.multiple_of(base + j * CHUNK, 8)

    @pl.when(c == 0)
    def _():
        _gather_pipe(lambda ig: tbl_lo.at[ig], gidx, sidx, acc_sh, cnt_sh,
                     aux, bufs, nsteps, off_fn)

    @pl.when(c == 1)
    def _():
        _gather_pipe(lambda ig: tbl_hi.at[ig], gidx, sidx, acc_sh, cnt_sh,
                     aux, bufs, nsteps, off_fn)

    plsc.subcore_barrier()
    _scale_writeback(acc_sh, cnt_sh, out_acc, c, s, r0, cbuf, aux)


def _scatter_phase2(veh16, pas_mean, dest, src, z16, z1):
    return pl.kernel(
        _scatter2_body,
        out_type=jax.ShapeDtypeStruct((2, N, 16), jnp.float32),
        mesh=_MESH,
        compiler_params=_SC_PARAMS,
        scratch_types=[
            pltpu.VMEM((CHUNK,), jnp.int32),
            pltpu.VMEM((CHUNK,), jnp.int32),
            pltpu.VMEM((CHUNK, 16), jnp.float32),
            pltpu.VMEM((CHUNK,), jnp.float32),
            pltpu.VMEM((CHUNK,), jnp.float32),
            pltpu.VMEM_SHARED((N, 16), jnp.float32),
            pltpu.VMEM_SHARED((N,), jnp.float32),
            pltpu.SemaphoreType.DMA,
        ],
    )(veh16, pas_mean, dest, src, z16, z1)


def kernel(requests_x, vehicles_x, passengers_x,
           veh2pas_receiver_edge_index, veh2pas_sender_edge_index,
           req2veh_sender_edge_index, req2veh_receiver_edge_index,
           W_req, b_req, W_veh, b_veh, W_pas, b_pas,
           W1, b1, W2, b2, W3, b3):
    veh16, pas16 = _encode(vehicles_x, passengers_x,
                           W_veh, b_veh.reshape(1, 16),
                           W_pas, b_pas.reshape(1, 16))

    z16 = jnp.zeros((N, 16), jnp.float32)
    z1 = jnp.zeros((N,), jnp.float32)

    meanpart = _scatter_phase1(
        pas16, veh2pas_receiver_edge_index, veh2pas_sender_edge_index, z16, z1)
    pas_mean = _merge(meanpart)

    mean2 = _scatter_phase2(
        veh16, pas_mean, req2veh_receiver_edge_index,
        req2veh_sender_edge_index, z16, z1)

    return _head(requests_x, mean2,
                 W_req, b_req.reshape(1, 16),
                 W1, b1.reshape(1, 64), W2, b2.reshape(1, 64),
                 W3, b3.reshape(1, 1))


# speedup vs baseline: 1.5857x; 1.0011x over previous
import functools
import jax
import jax.numpy as jnp
from jax import lax
from jax.experimental import pallas as pl
from jax.experimental.pallas import tpu as pltpu
from jax.experimental.pallas import tpu_sc as plsc

N = 100000
E = 1600000
ROW_BLK = 4000
CHUNK = 1000
NS = 16
ROWS_PER_TILE = 6256
ROWS_LAST = N - 15 * ROWS_PER_TILE
WB_CHUNK = 800

_SC_PARAMS = pltpu.CompilerParams(use_tc_tiling_on_sc=False)


def _mesh():
    return plsc.VectorSubcoreMesh(core_axis_name="c", subcore_axis_name="s")


def _encode_body(veh_x, pas_x, W_veh, b_veh, W_pas, b_pas, veh_o, pas_o):
    veh_o[...] = jnp.tanh(
        jnp.dot(veh_x[...], W_veh[...], preferred_element_type=jnp.float32) + b_veh[...])
    pas_o[...] = jnp.tanh(
        jnp.dot(pas_x[...], W_pas[...], preferred_element_type=jnp.float32) + b_pas[...])


def _encode(veh_x, pas_x, W_veh, b_veh, W_pas, b_pas):
    n = veh_x.shape[0]
    rows = lambda w: pl.BlockSpec((ROW_BLK, w), lambda i: (i, 0))
    full = lambda a: pl.BlockSpec(a.shape, lambda i: (0,) * a.ndim)
    return pl.pallas_call(
        _encode_body,
        grid=(n // ROW_BLK,),
        in_specs=[rows(8), rows(10),
                  full(W_veh), full(b_veh), full(W_pas), full(b_pas)],
        out_specs=[rows(16), rows(16)],
        out_shape=[jax.ShapeDtypeStruct((n, 16), jnp.float32)] * 2,
    )(veh_x, pas_x, W_veh, b_veh, W_pas, b_pas)


def _merge_body(a, b, out):
    out[...] = a[0] + b[0]


def _merge(acc):
    flat = acc.reshape(2, 12500, 128)
    out = pl.pallas_call(
        _merge_body,
        grid=(1,),
        in_specs=[pl.BlockSpec((1, 12500, 128), lambda i: (0, 0, 0)),
                  pl.BlockSpec((1, 12500, 128), lambda i: (1, 0, 0))],
        out_specs=pl.BlockSpec((12500, 128), lambda i: (0, 0)),
        out_shape=jax.ShapeDtypeStruct((12500, 128), jnp.float32),
    )(flat, flat)
    return out.reshape(N, 16)


def _head_body(req_x, m2, W_req, b_req, W1, b1, W2, b2, W3, b3, out):
    req = jnp.tanh(
        jnp.dot(req_x[...], W_req[...], preferred_element_type=jnp.float32) + b_req[...])
    act = jnp.concatenate([req, m2[0], m2[1]], axis=-1)
    h = jnp.tanh(jnp.dot(act, W1[...], preferred_element_type=jnp.float32) + b1[...])
    h = jnp.tanh(jnp.dot(h, W2[...], preferred_element_type=jnp.float32) + b2[...])
    out[...] = jnp.dot(h, W3[...], preferred_element_type=jnp.float32) + b3[...]


def _head(req_x, mean2, W_req, b_req, W1, b1, W2, b2, W3, b3):
    n = req_x.shape[0]
    rows = lambda w: pl.BlockSpec((ROW_BLK, w), lambda i: (i, 0))
    full = lambda a: pl.BlockSpec(a.shape, lambda i: (0,) * a.ndim)
    return pl.pallas_call(
        _head_body,
        grid=(n // ROW_BLK,),
        in_specs=[rows(10), pl.BlockSpec((2, ROW_BLK, 16), lambda i: (0, i, 0)),
                  full(W_req), full(b_req),
                  full(W1), full(b1), full(W2), full(b2), full(W3), full(b3)],
        out_specs=rows(1),
        out_shape=jax.ShapeDtypeStruct((n, 1), jnp.float32),
    )(req_x, mean2, W_req, b_req, W1, b1, W2, b2, W3, b3)


def _fill(ref, val):
    flat = ref.shape[0]
    v = jnp.full((16,), val, jnp.float32)

    def body(i, _):
        ref[pl.ds(i * 16, 16)] = v
        return 0

    lax.fori_loop(0, flat // 16, body, 0)
    if flat % 16:
        ref[pl.ds(flat - 16, 16)] = v


def _scale_rows(rows, cbuf, ibuf, nrows):
    def inv(i, _):
        ibuf[pl.ds(i * 16, 16)] = 1.0 / jnp.maximum(cbuf[pl.ds(i * 16, 16)], 1.0)
        return 0

    lax.fori_loop(0, nrows // 16, inv, 0)

    def mul16(k, _):
        iv = ibuf[pl.ds(k * 16, 16)]
        for j in range(16):
            rows[k * 16 + j, :] = rows[k * 16 + j, :] * iv[j]
        return 0

    lax.fori_loop(0, nrows // 16, mul16, 0)


def _scale_writeback(acc_sh, cnt_sh, out_acc, c, s, rows, cbuf, ibuf):
    base = pl.multiple_of(s * ROWS_PER_TILE, 8)

    def sub(k, _):
        r = pl.multiple_of(base + k * WB_CHUNK, 8)
        pltpu.sync_copy(acc_sh.at[pl.ds(r, WB_CHUNK)], rows.at[pl.ds(0, WB_CHUNK)])
        pltpu.sync_copy(cnt_sh.at[pl.ds(r, WB_CHUNK)], cbuf.at[pl.ds(0, WB_CHUNK)])
        _scale_rows(rows, cbuf, ibuf, WB_CHUNK)
        pltpu.sync_copy(rows.at[pl.ds(0, WB_CHUNK)], out_acc.at[c, pl.ds(r, WB_CHUNK)])
        return 0

    lax.fori_loop(0, 7, sub, 0)

    def tail(sz):
        r = pl.multiple_of(base + 7 * WB_CHUNK, 8)
        pltpu.sync_copy(acc_sh.at[pl.ds(r, sz)], rows.at[pl.ds(0, sz)])
        pltpu.sync_copy(cnt_sh.at[pl.ds(r, sz)], cbuf.at[pl.ds(0, sz)])
        _scale_rows(rows, cbuf, ibuf, sz)
        pltpu.sync_copy(rows.at[pl.ds(0, sz)], out_acc.at[c, pl.ds(r, sz)])

    @pl.when(s < NS - 1)
    def _():
        tail(ROWS_PER_TILE - 7 * WB_CHUNK)

    @pl.when(s == NS - 1)
    def _():
        tail(ROWS_LAST - 7 * WB_CHUNK)


def _gather_pipe(tbl_for, gidx, sidx, acc_sh, cnt_sh, aux,
                 bufs, nsteps, off_fn):
    ((ig0, is0, r0, sem0),) = bufs

    def step(j, _):
        off = off_fn(j)
        pltpu.sync_copy(gidx.at[pl.ds(off, CHUNK)], ig0)
        pltpu.sync_copy(sidx.at[pl.ds(off, CHUNK)], is0)
        pltpu.async_copy(tbl_for(ig0), r0, sem0).wait()
        pltpu.sync_copy(r0, acc_sh.at[is0], add=True)
        pltpu.sync_copy(aux, cnt_sh.at[is0], add=True)
        return 0

    lax.fori_loop(0, nsteps, step, 0)


def _count_pipe(sidx, cnt_sh, aux, bufs, nsteps, off_fn):
    ((is0, sem0),) = bufs

    def step(j, _):
        off = off_fn(j)
        pltpu.sync_copy(sidx.at[pl.ds(off, CHUNK)], is0)
        pltpu.sync_copy(aux, cnt_sh.at[is0], add=True)
        return 0

    lax.fori_loop(0, nsteps, step, 0)


def _scatter1_body(tbl, gidx, sidx, z16, z1, out_acc,
                   ig0, is0, r0, aux, cbuf, acc_sh, cnt_sh, sem0):
    c = lax.axis_index("c")
    s = lax.axis_index("s")
    _fill(aux, 1.0)

    @pl.when(s == 0)
    def _():
        pltpu.sync_copy(z16, acc_sh)
        pltpu.sync_copy(z1, cnt_sh)
    plsc.subcore_barrier()

    per_tile = E // NS
    feat_half = per_tile // 2
    cnt_base = s * per_tile
    nsteps = feat_half // CHUNK

    feat_base = cnt_base + c * feat_half
    _gather_pipe(lambda ig: tbl.at[ig], gidx, sidx, acc_sh, cnt_sh, aux,
                 [(ig0, is0, r0, sem0)], nsteps,
                 lambda j: pl.multiple_of(feat_base + j * CHUNK, 8))

    cnt_base2 = cnt_base + (1 - c) * feat_half
    _count_pipe(sidx, cnt_sh, aux, [(is0, sem0)], nsteps,
                lambda j: pl.multiple_of(cnt_base2 + j * CHUNK, 8))

    plsc.subcore_barrier()
    _scale_writeback(acc_sh, cnt_sh, out_acc, c, s, r0, cbuf, aux)


def _scatter_phase1(pas16, recv, send, z16, z1):
    return pl.kernel(
        _scatter1_body,
        out_type=jax.ShapeDtypeStruct((2, N, 16), jnp.float32),
        mesh=_mesh(),
        compiler_params=_SC_PARAMS,
        scratch_types=[
            pltpu.VMEM((CHUNK,), jnp.int32),
            pltpu.VMEM((CHUNK,), jnp.int32),
            pltpu.VMEM((CHUNK, 16), jnp.float32),
            pltpu.VMEM((CHUNK,), jnp.float32),
            pltpu.VMEM((CHUNK,), jnp.float32),
            pltpu.VMEM_SHARED((N, 16), jnp.float32),
            pltpu.VMEM_SHARED((N,), jnp.float32),
            pltpu.SemaphoreType.DMA,
        ],
    )(pas16, recv, send, z16, z1)


def _scatter2_body(tbl_lo, tbl_hi, gidx, sidx, z16, z1, out_acc,
                   ig0, is0, r0, aux, cbuf, acc_sh, cnt_sh, sem0):
    c = lax.axis_index("c")
    s = lax.axis_index("s")
    _fill(aux, 1.0)

    @pl.when(s == 0)
    def _():
        pltpu.sync_copy(z16, acc_sh)
        pltpu.sync_copy(z1, cnt_sh)
    plsc.subcore_barrier()

    per_tile = E // NS
    base = s * per_tile
    nsteps = per_tile // CHUNK
    bufs = [(ig0, is0, r0, sem0)]
    off_fn = lambda j: pl.multiple_of(base + j * CHUNK, 8)

    @pl.when(c == 0)
    def _():
        _gather_pipe(lambda ig: tbl_lo.at[ig], gidx, sidx, acc_sh, cnt_sh,
                     aux, bufs, nsteps, off_fn)

    @pl.when(c == 1)
    def _():
        _gather_pipe(lambda ig: tbl_hi.at[ig], gidx, sidx, acc_sh, cnt_sh,
                     aux, bufs, nsteps, off_fn)

    plsc.subcore_barrier()
    _scale_writeback(acc_sh, cnt_sh, out_acc, c, s, r0, cbuf, aux)


def _scatter_phase2(veh16, pas_mean, dest, src, z16, z1):
    return pl.kernel(
        _scatter2_body,
        out_type=jax.ShapeDtypeStruct((2, N, 16), jnp.float32),
        mesh=_mesh(),
        compiler_params=_SC_PARAMS,
        scratch_types=[
            pltpu.VMEM((CHUNK,), jnp.int32),
            pltpu.VMEM((CHUNK,), jnp.int32),
            pltpu.VMEM((CHUNK, 16), jnp.float32),
            pltpu.VMEM((CHUNK,), jnp.float32),
            pltpu.VMEM((CHUNK,), jnp.float32),
            pltpu.VMEM_SHARED((N, 16), jnp.float32),
            pltpu.VMEM_SHARED((N,), jnp.float32),
            pltpu.SemaphoreType.DMA,
        ],
    )(veh16, pas_mean, dest, src, z16, z1)


def kernel(requests_x, vehicles_x, passengers_x,
           veh2pas_receiver_edge_index, veh2pas_sender_edge_index,
           req2veh_sender_edge_index, req2veh_receiver_edge_index,
           W_req, b_req, W_veh, b_veh, W_pas, b_pas,
           W1, b1, W2, b2, W3, b3):
    veh16, pas16 = _encode(vehicles_x, passengers_x,
                           W_veh, b_veh.reshape(1, 16),
                           W_pas, b_pas.reshape(1, 16))

    z16 = jnp.zeros((N, 16), jnp.float32)
    z1 = jnp.zeros((N,), jnp.float32)

    meanpart = _scatter_phase1(
        pas16, veh2pas_receiver_edge_index, veh2pas_sender_edge_index, z16, z1)
    pas_mean = _merge(meanpart)

    mean2 = _scatter_phase2(
        veh16, pas_mean, req2veh_receiver_edge_index,
        req2veh_sender_edge_index, z16, z1)

    return _head(requests_x, mean2,
                 W_req, b_req.reshape(1, 16),
                 W1, b1.reshape(1, 64), W2, b2.reshape(1, 64),
                 W3, b3.reshape(1, 1))
